# baseline (speedup 1.0000x reference)
import functools

import jax
import jax.numpy as jnp
import numpy as np
from jax import lax
from jax.experimental import pallas as pl
from jax.experimental.pallas import tpu as pltpu
from jax.experimental.pallas import tpu_sc as plsc

_N = 10000
_NP = 10112
_E = 320000
_F = 16
_D = 128
_C = 80
_NCH = _E // _C
_NW = 32
_EPW = _E // _NW
_CPW = _NCH // _NW
_RPT = _NP // 16
_NB = 4

_HIM = np.int32(-65536)
_mesh = plsc.VectorSubcoreMesh(core_axis_name="c", subcore_axis_name="s")
_sc_params = pltpu.CompilerParams(use_tc_tiling_on_sc=False,
                                  needs_layout_passes=False)


@functools.partial(
    pl.kernel,
    mesh=_mesh,
    out_type=(
        jax.ShapeDtypeStruct((2, _NP, _F), jnp.float32),
        jax.ShapeDtypeStruct((2, _NP, _F), jnp.float32),
    ),
    scratch_types=[
        [pltpu.VMEM((_C, _F), jnp.float32) for _ in range(_NB)],
        [pltpu.VMEM((_C,), jnp.int32) for _ in range(_NB)],
        pltpu.VMEM((_C, _F), jnp.float32),
        pltpu.VMEM((_RPT, _F), jnp.float32),
        pltpu.VMEM_SHARED((_NP, _F), jnp.float32),
        pltpu.VMEM_SHARED((_NP, _F), jnp.float32),
        [pltpu.SemaphoreType.DMA for _ in range(_NB)],
        [pltpu.SemaphoreType.DMA for _ in range(_NB)],
        [pltpu.SemaphoreType.DMA for _ in range(_NB)],
        [pltpu.SemaphoreType.DMA for _ in range(_NB)],
    ],
    compiler_params=_sc_params,
)
def _sc_scatter(ef_hbm, ei_hbm, psum_hbm, pdeg_hbm,
                rv, iv, ones_v, ztile_v, acc_s, deg_s, srow, sidx, ssr, sso):
    cid = lax.axis_index("c")
    sid = lax.axis_index("s")
    wid = sid * 2 + cid

    zrow = jnp.zeros((16,), jnp.float32)
    one = jnp.ones((16,), jnp.float32)

    def _fill_ones(i, carry):
        ones_v[i, :] = one
        return carry
    lax.fori_loop(0, _C, _fill_ones, 0)

    def _zt(i, carry):
        ztile_v[i, :] = zrow
        return carry
    lax.fori_loop(0, _RPT, _zt, 0)

    pltpu.sync_copy(ztile_v, acc_s.at[pl.ds(sid * _RPT, _RPT)])
    pltpu.sync_copy(ztile_v, deg_s.at[pl.ds(sid * _RPT, _RPT)])
    plsc.subcore_barrier()

    def _stage(j, b):
        base = pl.multiple_of(wid * _EPW + j * _C, 8)
        return (pltpu.make_async_copy(ef_hbm.at[pl.ds(base, _C)], rv[b], srow[b]),
                pltpu.make_async_copy(ei_hbm.at[1, pl.ds(base, _C)], iv[b],
                                      sidx[b]))

    for j0 in range(2):
        sa, sb = _stage(j0, j0)
        sa.start()
        sb.start()

    def _step(j, b):
        sa, sb = _stage(j, b)
        sa.wait()
        sb.wait()

        b2 = (b + 2) % _NB

        @pl.when(j > 1)
        def _():
            pltpu.make_async_copy(rv[b2], acc_s.at[iv[b2]], ssr[b2]).wait()
            pltpu.make_async_copy(ones_v, deg_s.at[iv[b2]], sso[b2]).wait()

        pltpu.async_copy(rv[b], acc_s.at[iv[b]], ssr[b], add=True)
        pltpu.async_copy(ones_v, deg_s.at[iv[b]], sso[b], add=True)

        @pl.when(j + 2 < _CPW)
        def _():
            na, nb = _stage(j + 2, b2)
            na.start()
            nb.start()

    def _quad(t, carry):
        for q in range(_NB):
            _step(_NB * t + q, q)
        return carry
    lax.fori_loop(0, _CPW // _NB, _quad, 0)
    _step(_CPW - 1, (_CPW - 1) % _NB)

    for j in (_CPW - 2, _CPW - 1):
        b = j % _NB
        pltpu.make_async_copy(rv[b], acc_s.at[iv[b]], ssr[b]).wait()
        pltpu.make_async_copy(ones_v, deg_s.at[iv[b]], sso[b]).wait()

    plsc.subcore_barrier()

    pltpu.sync_copy(acc_s.at[pl.ds(sid * _RPT, _RPT)], ztile_v)
    pltpu.sync_copy(ztile_v, psum_hbm.at[cid, pl.ds(sid * _RPT, _RPT)])
    pltpu.sync_copy(deg_s.at[pl.ds(sid * _RPT, _RPT)], ztile_v)
    pltpu.sync_copy(ztile_v, pdeg_hbm.at[cid, pl.ds(sid * _RPT, _RPT)])


_DH = _D // 2
_CLO = np.empty((_DH,), dtype=np.int32)
_CHI = np.empty((_DH,), dtype=np.int32)
for _k in range(_DH):
    _g, _kp = _k // 16, _k % 16
    _CLO[_k] = 32 * _g + _kp
    _CHI[_k] = 32 * _g + 16 + _kp


def _bf16_bits(x):
    i = lax.bitcast_convert_type(x, jnp.int32)
    lsb = (i >> 16) & 1
    return ((i + 32767 + lsb) >> 16) & 0xFFFF


def _tc_linear_body(ps_ref, pd_ref, wn_ref, bn_ref, wal_ref, wah_ref,
                    wbl_ref, wbh_ref, bel_ref, beh_ref, a_ref, b_ref):
    s = ps_ref[0] + ps_ref[1]
    dg = jnp.maximum(pd_ref[0][:, :1] + pd_ref[1][:, :1], 1.0)
    h = s / dg
    h2 = jnp.maximum(
        jnp.dot(h, wn_ref[...], preferred_element_type=jnp.float32) + bn_ref[...],
        0.0)
    alo = jnp.dot(h2, wal_ref[...], preferred_element_type=jnp.float32)
    ahi = jnp.dot(h2, wah_ref[...], preferred_element_type=jnp.float32)
    blo = jnp.dot(h2, wbl_ref[...], preferred_element_type=jnp.float32) + bel_ref[...]
    bhi = jnp.dot(h2, wbh_ref[...], preferred_element_type=jnp.float32) + beh_ref[...]
    a_ref[...] = _bf16_bits(alo) | (_bf16_bits(ahi) << 16)
    b_ref[...] = _bf16_bits(blo) | (_bf16_bits(bhi) << 16)


_tc_linear = pl.pallas_call(
    _tc_linear_body,
    out_shape=(
        jax.ShapeDtypeStruct((_NP, _DH), jnp.int32),
        jax.ShapeDtypeStruct((_NP, _DH), jnp.int32),
    ),
)


@functools.partial(
    pl.kernel,
    mesh=_mesh,
    out_type=jax.ShapeDtypeStruct((_NCH, _C, _D), jnp.float32),
    scratch_types=[
        [pltpu.VMEM((_C,), jnp.int32) for _ in range(_NB)],
        [pltpu.VMEM((_C,), jnp.int32) for _ in range(_NB)],
        [pltpu.VMEM((_C, _DH), jnp.int32) for _ in range(_NB)],
        [pltpu.VMEM((_C, _DH), jnp.int32) for _ in range(_NB)],
        [pltpu.VMEM((_C, _D), jnp.float32) for _ in range(_NB)],
        [pltpu.SemaphoreType.DMA for _ in range(_NB)],
        [pltpu.SemaphoreType.DMA for _ in range(_NB)],
        [pltpu.SemaphoreType.DMA for _ in range(_NB)],
        [pltpu.SemaphoreType.DMA for _ in range(_NB)],
    ],
    compiler_params=_sc_params,
)
def _sc_gather(a_hbm, b_hbm, ei_hbm, out_hbm,
               isr, idr, ra, rb, ov, si, sga, sgb, so):
    cid = lax.axis_index("c")
    sid = lax.axis_index("s")
    wid = sid * 2 + cid

    def _idx(i, b):
        base = pl.multiple_of(wid * _EPW + i * _C, 8)
        return (pltpu.make_async_copy(ei_hbm.at[0, pl.ds(base, _C)], isr[b],
                                      si[b]),
                pltpu.make_async_copy(ei_hbm.at[1, pl.ds(base, _C)], idr[b],
                                      si[b]))

    def _gath(b):
        return (pltpu.make_async_copy(a_hbm.at[isr[b]], ra[b], sga[b]),
                pltpu.make_async_copy(b_hbm.at[idr[b]], rb[b], sgb[b]))

    def _out(i, b):
        return pltpu.make_async_copy(ov[b], out_hbm.at[wid * _CPW + i], so[b])

    for j0 in range(_NB):
        pa, pb = _idx(j0, j0)
        pa.start()
        pb.start()
    for j0 in range(3):
        pa, pb = _idx(j0, j0)
        pa.wait()
        pb.wait()
        ga, gb = _gath(j0)
        ga.start()
        gb.start()

    def _half(i, b):
        ga_, gb_ = _gath(b)
        ga_.wait()
        gb_.wait()

        b3 = (b + 3) % _NB

        @pl.when(i + 3 < _CPW)
        def _():
            wa_, wb_ = _idx(i + 3, b3)
            wa_.wait()
            wb_.wait()

        @pl.when(i > 0)
        def _():
            _out(i - 1, b3).wait()

        @pl.when(i + 3 < _CPW)
        def _():
            na, nb = _gath(b3)
            na.start()
            nb.start()

        @pl.when(i + 4 < _CPW)
        def _():
            pa, pb = _idx(i + 4, b)
            pa.start()
            pb.start()

        def _addrow(k, c2):
            for g in range(_DH // 16):
                sl = pl.ds(g * 16, 16)
                wa = ra[b][k, sl]
                wb = rb[b][k, sl]
                lo = (plsc.bitcast(wa << 16, jnp.float32)
                      + plsc.bitcast(wb << 16, jnp.float32))
                hi = (plsc.bitcast(wa & _HIM, jnp.float32)
                      + plsc.bitcast(wb & _HIM, jnp.float32))
                ov[b][k, pl.ds(g * 32, 16)] = lo
                ov[b][k, pl.ds(g * 32 + 16, 16)] = hi
            return c2
        lax.fori_loop(0, _C, _addrow, 0)
        _out(i, b).start()

    def _quad(t, carry):
        for q in range(_NB):
            _half(_NB * t + q, q)
        return carry
    lax.fori_loop(0, _CPW // _NB, _quad, 0)
    _half(_CPW - 1, (_CPW - 1) % _NB)
    _out(_CPW - 1, (_CPW - 1) % _NB).wait()


def kernel(nfeats, efeats, edge_index, W_neigh1, b_neigh1, W_edge1, b_edge1,
           W_neigh2, b_neigh2, W_edge2, b_edge2):
    ei = edge_index.astype(jnp.int32)
    clo = jnp.asarray(_CLO)
    chi = jnp.asarray(_CHI)
    wa = W_edge2[:, :_D].T
    wb = W_edge2[:, _D:].T
    psum, pdeg = _sc_scatter(efeats, ei)
    a_tab, b_tab = _tc_linear(
        psum, pdeg,
        W_neigh2.T, b_neigh2.reshape(1, _D),
        wa[:, clo], wa[:, chi], wb[:, clo], wb[:, chi],
        b_edge2[clo].reshape(1, _DH), b_edge2[chi].reshape(1, _DH))
    out = _sc_gather(a_tab, b_tab, ei)
    return out.reshape(_E, _D)


# speedup vs baseline: 1.2084x; 1.2084x over previous
import functools

import jax
import jax.numpy as jnp
from jax import lax
from jax.experimental import pallas as pl
from jax.experimental.pallas import tpu as pltpu
from jax.experimental.pallas import tpu_sc as plsc

_N = 10000
_NP = 10112
_E = 320000
_F = 16
_D = 128
_C = 80
_NCH = _E // _C
_NW = 32
_EPW = _E // _NW
_CPW = _NCH // _NW
_RPT = _NP // 16
_NB = 4
_U = 128
_NU = _E // _U
_UPW = _NU // _NW

_mesh = plsc.VectorSubcoreMesh(core_axis_name="c", subcore_axis_name="s")
_sc_params = pltpu.CompilerParams(use_tc_tiling_on_sc=False,
                                  needs_layout_passes=False)


@functools.partial(
    pl.kernel,
    mesh=_mesh,
    out_type=(
        jax.ShapeDtypeStruct((2, _NP, _F), jnp.float32),
        jax.ShapeDtypeStruct((2, _NP, _F), jnp.float32),
    ),
    scratch_types=[
        [pltpu.VMEM((2, 8, 8, _F), jnp.float32) for _ in range(_NB)],
        [pltpu.VMEM((_U, _F), jnp.float32) for _ in range(_NB)],
        [pltpu.VMEM((_U,), jnp.int32) for _ in range(_NB)],
        pltpu.VMEM((_U, _F), jnp.float32),
        pltpu.VMEM((_RPT, _F), jnp.float32),
        pltpu.VMEM_SHARED((_NP, _F), jnp.float32),
        pltpu.VMEM_SHARED((_NP, _F), jnp.float32),
        [pltpu.SemaphoreType.DMA for _ in range(_NB)],
        [pltpu.SemaphoreType.DMA for _ in range(_NB)],
        [pltpu.SemaphoreType.DMA for _ in range(_NB)],
        [pltpu.SemaphoreType.DMA for _ in range(_NB)],
    ],
    compiler_params=_sc_params,
)
def _sc_scatter(ef_hbm, ei_hbm, psum_hbm, pdeg_hbm,
                rt, rv, iv, ones_v, ztile_v, acc_s, deg_s, srow, sidx, ssr, sso):
    cid = lax.axis_index("c")
    sid = lax.axis_index("s")
    wid = sid * 2 + cid
    nsteps = _UPW + jnp.where(wid < _NU - _NW * _UPW, 1, 0)

    zrow = jnp.zeros((16,), jnp.float32)
    one = jnp.ones((16,), jnp.float32)

    def _fill_ones(i, carry):
        ones_v[i, :] = one
        return carry
    lax.fori_loop(0, _U, _fill_ones, 0)

    def _zt(i, carry):
        ztile_v[i, :] = zrow
        return carry
    lax.fori_loop(0, _RPT, _zt, 0)

    pltpu.sync_copy(ztile_v, acc_s.at[pl.ds(sid * _RPT, _RPT)])
    pltpu.sync_copy(ztile_v, deg_s.at[pl.ds(sid * _RPT, _RPT)])
    plsc.subcore_barrier()

    def _unit(j):
        return jnp.where(j >= _UPW, _NW * _UPW + wid, wid * _UPW + j)

    def _stage(j, b):
        u = _unit(j)
        return (pltpu.make_async_copy(ef_hbm.at[pl.ds(2 * u, 2)], rt[b],
                                      srow[b]),
                pltpu.make_async_copy(
                    ei_hbm.at[1, pl.ds(pl.multiple_of(_U * u, 8), _U)],
                    iv[b], sidx[b]))

    for j0 in range(2):
        sa, sb = _stage(j0, j0)
        sa.start()
        sb.start()

    def _step(j, b):
        sa, sb = _stage(j, b)
        sa.wait()
        sb.wait()

        for q in range(2):
            for s in range(8):
                for g in range(8):
                    rv[b][64 * q + 8 * g + s, :] = rt[b][q, s, g, :]

        b2 = (b + 2) % _NB

        @pl.when(j > 1)
        def _():
            pltpu.make_async_copy(rv[b2], acc_s.at[iv[b2]], ssr[b2]).wait()
            pltpu.make_async_copy(ones_v, deg_s.at[iv[b2]], sso[b2]).wait()

        pltpu.async_copy(rv[b], acc_s.at[iv[b]], ssr[b], add=True)
        pltpu.async_copy(ones_v, deg_s.at[iv[b]], sso[b], add=True)

        @pl.when(j + 2 < nsteps)
        def _():
            na, nb = _stage(j + 2, b2)
            na.start()
            nb.start()

    def _quad(t, carry):
        for q in range(_NB):
            _step(_NB * t + q, q)
        return carry
    lax.fori_loop(0, _UPW // _NB, _quad, 0)
    _step(_UPW - 2, (_UPW - 2) % _NB)
    _step(_UPW - 1, (_UPW - 1) % _NB)

    def _drain(j):
        b = j % _NB
        pltpu.make_async_copy(rv[b], acc_s.at[iv[b]], ssr[b]).wait()
        pltpu.make_async_copy(ones_v, deg_s.at[iv[b]], sso[b]).wait()

    @pl.when(nsteps > _UPW)
    def _():
        _step(_UPW, _UPW % _NB)
        _drain(_UPW)

    @pl.when(nsteps == _UPW)
    def _():
        _drain(_UPW - 2)

    _drain(_UPW - 1)

    plsc.subcore_barrier()

    pltpu.sync_copy(acc_s.at[pl.ds(sid * _RPT, _RPT)], ztile_v)
    pltpu.sync_copy(ztile_v, psum_hbm.at[cid, pl.ds(sid * _RPT, _RPT)])
    pltpu.sync_copy(deg_s.at[pl.ds(sid * _RPT, _RPT)], ztile_v)
    pltpu.sync_copy(ztile_v, pdeg_hbm.at[cid, pl.ds(sid * _RPT, _RPT)])


def _tc_linear_body(ps_ref, pd_ref, wn_ref, bn_ref, wa_ref, wb_ref, be_ref,
                    a_ref, b_ref):
    s = ps_ref[0] + ps_ref[1]
    dg = jnp.maximum(pd_ref[0][:, :1] + pd_ref[1][:, :1], 1.0)
    h = s / dg
    h2 = jnp.maximum(
        jnp.dot(h, wn_ref[...], preferred_element_type=jnp.float32) + bn_ref[...],
        0.0)
    a_ref[...] = jnp.dot(h2, wa_ref[...], preferred_element_type=jnp.float32)
    b_ref[...] = (jnp.dot(h2, wb_ref[...], preferred_element_type=jnp.float32)
                  + be_ref[...])


_tc_linear = pl.pallas_call(
    _tc_linear_body,
    out_shape=(
        jax.ShapeDtypeStruct((_NP, _D), jnp.float32),
        jax.ShapeDtypeStruct((_NP, _D), jnp.float32),
    ),
)


@functools.partial(
    pl.kernel,
    mesh=_mesh,
    out_type=jax.ShapeDtypeStruct((_NCH, _C, _D), jnp.float32),
    scratch_types=[
        [pltpu.VMEM((_C,), jnp.int32) for _ in range(_NB)],
        [pltpu.VMEM((_C,), jnp.int32) for _ in range(_NB)],
        [pltpu.VMEM((_C, _D), jnp.float32) for _ in range(_NB)],
        [pltpu.VMEM((_C, _D), jnp.float32) for _ in range(_NB)],
        [pltpu.SemaphoreType.DMA for _ in range(_NB)],
        [pltpu.SemaphoreType.DMA for _ in range(_NB)],
        [pltpu.SemaphoreType.DMA for _ in range(_NB)],
        [pltpu.SemaphoreType.DMA for _ in range(_NB)],
    ],
    compiler_params=_sc_params,
)
def _sc_gather(a_hbm, b_hbm, ei_hbm, out_hbm,
               isr, idr, ra, rb, si, sga, sgb, so):
    cid = lax.axis_index("c")
    sid = lax.axis_index("s")
    wid = sid * 2 + cid

    def _idx(i, b):
        base = pl.multiple_of(wid * _EPW + i * _C, 8)
        return (pltpu.make_async_copy(ei_hbm.at[0, pl.ds(base, _C)], isr[b],
                                      si[b]),
                pltpu.make_async_copy(ei_hbm.at[1, pl.ds(base, _C)], idr[b],
                                      si[b]))

    def _gath(b):
        return (pltpu.make_async_copy(a_hbm.at[isr[b]], ra[b], sga[b]),
                pltpu.make_async_copy(b_hbm.at[idr[b]], rb[b], sgb[b]))

    def _out(i, b):
        return pltpu.make_async_copy(rb[b], out_hbm.at[wid * _CPW + i], so[b])

    for j0 in range(_NB):
        pa, pb = _idx(j0, j0)
        pa.start()
        pb.start()
    for j0 in range(3):
        pa, pb = _idx(j0, j0)
        pa.wait()
        pb.wait()
        ga, gb = _gath(j0)
        ga.start()
        gb.start()

    def _half(i, b):
        ga_, gb_ = _gath(b)
        ga_.wait()
        gb_.wait()

        b3 = (b + 3) % _NB

        @pl.when(i + 3 < _CPW)
        def _():
            wa_, wb_ = _idx(i + 3, b3)
            wa_.wait()
            wb_.wait()

        @pl.when(i > 0)
        def _():
            _out(i - 1, b3).wait()

        @pl.when(i + 3 < _CPW)
        def _():
            na, nb = _gath(b3)
            na.start()
            nb.start()

        @pl.when(i + 4 < _CPW)
        def _():
            pa, pb = _idx(i + 4, b)
            pa.start()
            pb.start()

        def _addrow(k, c2):
            for g in range(_D // 16):
                sl = pl.ds(g * 16, 16)
                rb[b][k, sl] += ra[b][k, sl]
            return c2
        lax.fori_loop(0, _C, _addrow, 0)
        _out(i, b).start()

    def _quad(t, carry):
        for q in range(_NB):
            _half(_NB * t + q, q)
        return carry
    lax.fori_loop(0, _CPW // _NB, _quad, 0)
    _half(_CPW - 1, (_CPW - 1) % _NB)
    _out(_CPW - 1, (_CPW - 1) % _NB).wait()


def kernel(nfeats, efeats, edge_index, W_neigh1, b_neigh1, W_edge1, b_edge1,
           W_neigh2, b_neigh2, W_edge2, b_edge2):
    ei = edge_index.astype(jnp.int32)
    ef_t = efeats.reshape(_E // 64, 8, 8, _F).transpose(0, 2, 1, 3)
    ef_t = ef_t.reshape(_NU * 2, 8, 8, _F)
    psum, pdeg = _sc_scatter(ef_t, ei)
    a_tab, b_tab = _tc_linear(
        psum, pdeg,
        W_neigh2.T, b_neigh2.reshape(1, _D),
        W_edge2[:, :_D].T, W_edge2[:, _D:].T, b_edge2.reshape(1, _D))
    out = _sc_gather(a_tab, b_tab, ei)
    return out.reshape(_E, _D)


# speedup vs baseline: 1.3642x; 1.1289x over previous
import functools

import jax
import jax.numpy as jnp
from jax import lax
from jax.experimental import pallas as pl
from jax.experimental.pallas import tpu as pltpu
from jax.experimental.pallas import tpu_sc as plsc

_N = 10000
_NP = 10112
_E = 320000
_F = 16
_D = 128
_C = 80
_NCH = _E // _C
_NW = 32
_EPW = _E // _NW
_CPW = _NCH // _NW
_RPT = _NP // 16
_NB = 4

_mesh = plsc.VectorSubcoreMesh(core_axis_name="c", subcore_axis_name="s")
_sc_params = pltpu.CompilerParams(use_tc_tiling_on_sc=False,
                                  needs_layout_passes=False)


@functools.partial(
    pl.kernel,
    mesh=_mesh,
    out_type=(
        jax.ShapeDtypeStruct((2, _NP, _F), jnp.float32),
        jax.ShapeDtypeStruct((2, _NP, _F), jnp.float32),
    ),
    scratch_types=[
        [pltpu.VMEM((_C, _F), jnp.float32) for _ in range(_NB)],
        [pltpu.VMEM((_C,), jnp.int32) for _ in range(_NB)],
        pltpu.VMEM((_C, _F), jnp.float32),
        pltpu.VMEM((_RPT, _F), jnp.float32),
        pltpu.VMEM_SHARED((_NP, _F), jnp.float32),
        pltpu.VMEM_SHARED((_NP, _F), jnp.float32),
        [pltpu.SemaphoreType.DMA for _ in range(_NB)],
        [pltpu.SemaphoreType.DMA for _ in range(_NB)],
        [pltpu.SemaphoreType.DMA for _ in range(_NB)],
        [pltpu.SemaphoreType.DMA for _ in range(_NB)],
    ],
    compiler_params=_sc_params,
)
def _sc_scatter(ef_hbm, ei_hbm, psum_hbm, pdeg_hbm,
                rv, iv, ones_v, ztile_v, acc_s, deg_s, srow, sidx, ssr, sso):
    cid = lax.axis_index("c")
    sid = lax.axis_index("s")
    wid = sid * 2 + cid

    zrow = jnp.zeros((16,), jnp.float32)
    one = jnp.ones((16,), jnp.float32)

    def _fill_ones(i, carry):
        ones_v[i, :] = one
        return carry
    lax.fori_loop(0, _C, _fill_ones, 0)

    def _zt(i, carry):
        ztile_v[i, :] = zrow
        return carry
    lax.fori_loop(0, _RPT, _zt, 0)

    pltpu.sync_copy(ztile_v, acc_s.at[pl.ds(sid * _RPT, _RPT)])
    pltpu.sync_copy(ztile_v, deg_s.at[pl.ds(sid * _RPT, _RPT)])
    plsc.subcore_barrier()

    def _stage(j, b):
        base = pl.multiple_of(wid * _EPW + j * _C, 8)
        return (pltpu.make_async_copy(ef_hbm.at[pl.ds(base, _C)], rv[b], srow[b]),
                pltpu.make_async_copy(ei_hbm.at[1, pl.ds(base, _C)], iv[b],
                                      sidx[b]))

    for j0 in range(2):
        sa, sb = _stage(j0, j0)
        sa.start()
        sb.start()

    def _step(j, b):
        sa, sb = _stage(j, b)
        sa.wait()
        sb.wait()

        b2 = (b + 2) % _NB

        @pl.when(j > 1)
        def _():
            pltpu.make_async_copy(rv[b2], acc_s.at[iv[b2]], ssr[b2]).wait()
            pltpu.make_async_copy(ones_v, deg_s.at[iv[b2]], sso[b2]).wait()

        pltpu.async_copy(rv[b], acc_s.at[iv[b]], ssr[b], add=True)
        pltpu.async_copy(ones_v, deg_s.at[iv[b]], sso[b], add=True)

        @pl.when(j + 2 < _CPW)
        def _():
            na, nb = _stage(j + 2, b2)
            na.start()
            nb.start()

    def _quad(t, carry):
        for q in range(_NB):
            _step(_NB * t + q, q)
        return carry
    lax.fori_loop(0, _CPW // _NB, _quad, 0)
    _step(_CPW - 1, (_CPW - 1) % _NB)

    for j in (_CPW - 2, _CPW - 1):
        b = j % _NB
        pltpu.make_async_copy(rv[b], acc_s.at[iv[b]], ssr[b]).wait()
        pltpu.make_async_copy(ones_v, deg_s.at[iv[b]], sso[b]).wait()

    plsc.subcore_barrier()

    pltpu.sync_copy(acc_s.at[pl.ds(sid * _RPT, _RPT)], ztile_v)
    pltpu.sync_copy(ztile_v, psum_hbm.at[cid, pl.ds(sid * _RPT, _RPT)])
    pltpu.sync_copy(deg_s.at[pl.ds(sid * _RPT, _RPT)], ztile_v)
    pltpu.sync_copy(ztile_v, pdeg_hbm.at[cid, pl.ds(sid * _RPT, _RPT)])


def _tc_linear_body(ps_ref, pd_ref, wn_ref, bn_ref, wa_ref, wb_ref, be_ref,
                    a_ref, b_ref):
    s = ps_ref[0] + ps_ref[1]
    dg = jnp.maximum(pd_ref[0][:, :1] + pd_ref[1][:, :1], 1.0)
    h = s / dg
    h2 = jnp.maximum(
        jnp.dot(h, wn_ref[...], preferred_element_type=jnp.float32) + bn_ref[...],
        0.0)
    a_ref[...] = jnp.dot(h2, wa_ref[...], preferred_element_type=jnp.float32)
    b_ref[...] = (jnp.dot(h2, wb_ref[...], preferred_element_type=jnp.float32)
                  + be_ref[...])


_tc_linear = pl.pallas_call(
    _tc_linear_body,
    out_shape=(
        jax.ShapeDtypeStruct((_NP, _D), jnp.float32),
        jax.ShapeDtypeStruct((_NP, _D), jnp.float32),
    ),
)


@functools.partial(
    pl.kernel,
    mesh=_mesh,
    out_type=jax.ShapeDtypeStruct((_NCH, _C, _D), jnp.float32),
    scratch_types=[
        [pltpu.VMEM((_C,), jnp.int32) for _ in range(_NB)],
        [pltpu.VMEM((_C,), jnp.int32) for _ in range(_NB)],
        [pltpu.VMEM((_C, _D), jnp.float32) for _ in range(_NB)],
        [pltpu.VMEM((_C, _D), jnp.float32) for _ in range(_NB)],
        [pltpu.SemaphoreType.DMA for _ in range(_NB)],
        [pltpu.SemaphoreType.DMA for _ in range(_NB)],
        [pltpu.SemaphoreType.DMA for _ in range(_NB)],
        [pltpu.SemaphoreType.DMA for _ in range(_NB)],
    ],
    compiler_params=_sc_params,
)
def _sc_gather(a_hbm, b_hbm, ei_hbm, out_hbm,
               isr, idr, ra, rb, si, sga, sgb, so):
    cid = lax.axis_index("c")
    sid = lax.axis_index("s")
    wid = sid * 2 + cid

    def _idx(i, b):
        base = pl.multiple_of(wid * _EPW + i * _C, 8)
        return (pltpu.make_async_copy(ei_hbm.at[0, pl.ds(base, _C)], isr[b],
                                      si[b]),
                pltpu.make_async_copy(ei_hbm.at[1, pl.ds(base, _C)], idr[b],
                                      si[b]))

    def _gath(b):
        return (pltpu.make_async_copy(a_hbm.at[isr[b]], ra[b], sga[b]),
                pltpu.make_async_copy(b_hbm.at[idr[b]], rb[b], sgb[b]))

    def _out(i, b):
        return pltpu.make_async_copy(rb[b], out_hbm.at[wid * _CPW + i], so[b])

    for j0 in range(_NB):
        pa, pb = _idx(j0, j0)
        pa.start()
        pb.start()
    for j0 in range(3):
        pa, pb = _idx(j0, j0)
        pa.wait()
        pb.wait()
        ga, gb = _gath(j0)
        ga.start()
        gb.start()

    def _half(i, b):
        ga_, gb_ = _gath(b)
        ga_.wait()
        gb_.wait()

        b3 = (b + 3) % _NB

        @pl.when(i + 3 < _CPW)
        def _():
            wa_, wb_ = _idx(i + 3, b3)
            wa_.wait()
            wb_.wait()

        @pl.when(i > 0)
        def _():
            _out(i - 1, b3).wait()

        @pl.when(i + 3 < _CPW)
        def _():
            na, nb = _gath(b3)
            na.start()
            nb.start()

        @pl.when(i + 4 < _CPW)
        def _():
            pa, pb = _idx(i + 4, b)
            pa.start()
            pb.start()

        def _addrow(k, c2):
            for g in range(_D // 16):
                sl = pl.ds(g * 16, 16)
                rb[b][k, sl] += ra[b][k, sl]
            return c2
        lax.fori_loop(0, _C, _addrow, 0)
        _out(i, b).start()

    def _quad(t, carry):
        for q in range(_NB):
            _half(_NB * t + q, q)
        return carry
    lax.fori_loop(0, _CPW // _NB, _quad, 0)
    _half(_CPW - 1, (_CPW - 1) % _NB)
    _out(_CPW - 1, (_CPW - 1) % _NB).wait()


def kernel(nfeats, efeats, edge_index, W_neigh1, b_neigh1, W_edge1, b_edge1,
           W_neigh2, b_neigh2, W_edge2, b_edge2):
    ei = edge_index.astype(jnp.int32)
    psum, pdeg = _sc_scatter(efeats, ei)
    a_tab, b_tab = _tc_linear(
        psum, pdeg,
        W_neigh2.T, b_neigh2.reshape(1, _D),
        W_edge2[:, :_D].T, W_edge2[:, _D:].T, b_edge2.reshape(1, _D))
    out = _sc_gather(a_tab, b_tab, ei)
    return out.reshape(_E, _D)


# speedup vs baseline: 1.4353x; 1.0522x over previous
import functools

import jax
import jax.numpy as jnp
from jax import lax
from jax.experimental import pallas as pl
from jax.experimental.pallas import tpu as pltpu
from jax.experimental.pallas import tpu_sc as plsc

_N = 10000
_NP = 10112
_E = 320000
_F = 16
_D = 128
_C = 80
_NCH = _E // _C
_NW = 32
_EPW = _E // _NW
_CPW = _NCH // _NW
_RPT = _NP // 16
_SUB = 25
_SUP = _SUB * _C
_NSUP = _E // _SUP
_SPW = _NSUP // _NW
_NB = 4

_mesh = plsc.VectorSubcoreMesh(core_axis_name="c", subcore_axis_name="s")
_sc_params = pltpu.CompilerParams(use_tc_tiling_on_sc=False,
                                  needs_layout_passes=False)


@functools.partial(
    pl.kernel,
    mesh=_mesh,
    out_type=(
        jax.ShapeDtypeStruct((2, _NP, _F), jnp.float32),
        jax.ShapeDtypeStruct((2, _NP, _F), jnp.float32),
    ),
    scratch_types=[
        pltpu.VMEM((_CPW, _C), jnp.int32),
        pltpu.VMEM((_SUP, _F), jnp.float32),
        pltpu.VMEM((_SUP, _F), jnp.float32),
        pltpu.VMEM((_C, _F), jnp.float32),
        pltpu.VMEM((_RPT, _F), jnp.float32),
        pltpu.VMEM_SHARED((_NP, _F), jnp.float32),
        pltpu.VMEM_SHARED((_NP, _F), jnp.float32),
        pltpu.SemaphoreType.DMA,
        pltpu.SemaphoreType.DMA,
        pltpu.SemaphoreType.DMA,
        pltpu.SemaphoreType.DMA,
    ],
    compiler_params=_sc_params,
)
def _sc_scatter(ef_hbm, dst3_hbm, psum_hbm, pdeg_hbm,
                dall_v, rv0, rv1, ones_v, ztile_v, acc_s, deg_s,
                srow0, srow1, ssr, sso):
    cid = lax.axis_index("c")
    sid = lax.axis_index("s")
    wid = sid * 2 + cid
    rv = (rv0, rv1)
    srow = (srow0, srow1)

    zrow = jnp.zeros((16,), jnp.float32)
    one = jnp.ones((16,), jnp.float32)

    def _fill_ones(i, carry):
        ones_v[i, :] = one
        return carry
    lax.fori_loop(0, _C, _fill_ones, 0)

    def _zt(i, carry):
        ztile_v[i, :] = zrow
        return carry
    lax.fori_loop(0, _RPT, _zt, 0)

    pltpu.sync_copy(ztile_v, acc_s.at[pl.ds(sid * _RPT, _RPT)])
    pltpu.sync_copy(ztile_v, deg_s.at[pl.ds(sid * _RPT, _RPT)])
    plsc.subcore_barrier()

    pltpu.sync_copy(dst3_hbm.at[wid], dall_v)

    def _rows_copy(s, b):
        base = pl.multiple_of((wid * _SPW + s) * _SUP, 8)
        return pltpu.make_async_copy(ef_hbm.at[pl.ds(base, _SUP)], rv[b], srow[b])

    _rows_copy(0, 0).start()
    _rows_copy(1, 1).start()

    def _scat_rows(b, j, row):
        return (rv[b].at[pl.ds(j * _C, _C)], acc_s.at[dall_v.at[row]], ssr)

    for s in range(_SPW):
        b = s % 2
        _rows_copy(s, b).wait()

        def _fire(j, carry):
            row = s * _SUB + j
            src, dst, sem = _scat_rows(b, j, row)
            pltpu.async_copy(src, dst, sem, add=True)
            pltpu.async_copy(ones_v, deg_s.at[dall_v.at[row]], sso, add=True)
            return carry
        lax.fori_loop(0, _SUB, _fire, 0)

        def _drain(j, carry):
            src, dst, sem = _scat_rows(b, j, s * _SUB + j)
            pltpu.make_async_copy(src, dst, sem).wait()
            return carry
        lax.fori_loop(0, _SUB, _drain, 0)

        if s + 2 < _SPW:
            _rows_copy(s + 2, b).start()

    def _drain_ones(j, carry):
        pltpu.make_async_copy(ones_v, deg_s.at[dall_v.at[j]], sso).wait()
        return carry
    lax.fori_loop(0, _CPW, _drain_ones, 0)

    plsc.subcore_barrier()

    pltpu.sync_copy(acc_s.at[pl.ds(sid * _RPT, _RPT)], ztile_v)
    pltpu.sync_copy(ztile_v, psum_hbm.at[cid, pl.ds(sid * _RPT, _RPT)])
    pltpu.sync_copy(deg_s.at[pl.ds(sid * _RPT, _RPT)], ztile_v)
    pltpu.sync_copy(ztile_v, pdeg_hbm.at[cid, pl.ds(sid * _RPT, _RPT)])


def _tc_linear_body(ps_ref, pd_ref, wn_ref, bn_ref, wa_ref, wb_ref, be_ref,
                    a_ref, b_ref):
    s = ps_ref[0] + ps_ref[1]
    dg = jnp.maximum(pd_ref[0][:, :1] + pd_ref[1][:, :1], 1.0)
    h = s / dg
    h2 = jnp.maximum(
        jnp.dot(h, wn_ref[...], preferred_element_type=jnp.float32) + bn_ref[...],
        0.0)
    a_ref[...] = jnp.dot(h2, wa_ref[...], preferred_element_type=jnp.float32)
    b_ref[...] = (jnp.dot(h2, wb_ref[...], preferred_element_type=jnp.float32)
                  + be_ref[...])


_tc_linear = pl.pallas_call(
    _tc_linear_body,
    out_shape=(
        jax.ShapeDtypeStruct((_NP, _D), jnp.float32),
        jax.ShapeDtypeStruct((_NP, _D), jnp.float32),
    ),
)


@functools.partial(
    pl.kernel,
    mesh=_mesh,
    out_type=jax.ShapeDtypeStruct((_NCH, _C, _D), jnp.float32),
    scratch_types=[
        [pltpu.VMEM((_C,), jnp.int32) for _ in range(_NB)],
        [pltpu.VMEM((_C,), jnp.int32) for _ in range(_NB)],
        [pltpu.VMEM((_C, _D), jnp.float32) for _ in range(_NB)],
        [pltpu.VMEM((_C, _D), jnp.float32) for _ in range(_NB)],
        [pltpu.SemaphoreType.DMA for _ in range(_NB)],
        [pltpu.SemaphoreType.DMA for _ in range(_NB)],
        [pltpu.SemaphoreType.DMA for _ in range(_NB)],
        [pltpu.SemaphoreType.DMA for _ in range(_NB)],
    ],
    compiler_params=_sc_params,
)
def _sc_gather(a_hbm, b_hbm, ei_hbm, out_hbm,
               isr, idr, ra, rb, si, sga, sgb, so):
    cid = lax.axis_index("c")
    sid = lax.axis_index("s")
    wid = sid * 2 + cid

    def _idx(i, b):
        base = pl.multiple_of(wid * _EPW + i * _C, 8)
        return (pltpu.make_async_copy(ei_hbm.at[0, pl.ds(base, _C)], isr[b],
                                      si[b]),
                pltpu.make_async_copy(ei_hbm.at[1, pl.ds(base, _C)], idr[b],
                                      si[b]))

    def _gath(b):
        return (pltpu.make_async_copy(a_hbm.at[isr[b]], ra[b], sga[b]),
                pltpu.make_async_copy(b_hbm.at[idr[b]], rb[b], sgb[b]))

    def _out(i, b):
        return pltpu.make_async_copy(rb[b], out_hbm.at[wid * _CPW + i], so[b])

    for j0 in range(_NB):
        pa, pb = _idx(j0, j0)
        pa.start()
        pb.start()
    for j0 in range(3):
        pa, pb = _idx(j0, j0)
        pa.wait()
        pb.wait()
        ga, gb = _gath(j0)
        ga.start()
        gb.start()

    def _half(i, b):
        ga_, gb_ = _gath(b)
        ga_.wait()
        gb_.wait()

        b3 = (b + 3) % _NB

        @pl.when(i + 3 < _CPW)
        def _():
            wa_, wb_ = _idx(i + 3, b3)
            wa_.wait()
            wb_.wait()

        @pl.when(i > 0)
        def _():
            _out(i - 1, b3).wait()

        @pl.when(i + 3 < _CPW)
        def _():
            na, nb = _gath(b3)
            na.start()
            nb.start()

        @pl.when(i + 4 < _CPW)
        def _():
            pa, pb = _idx(i + 4, b)
            pa.start()
            pb.start()

        def _addrow(k, c2):
            for g in range(_D // 16):
                sl = pl.ds(g * 16, 16)
                rb[b][k, sl] += ra[b][k, sl]
            return c2
        lax.fori_loop(0, _C, _addrow, 0)
        _out(i, b).start()

    def _quad(t, carry):
        for q in range(_NB):
            _half(_NB * t + q, q)
        return carry
    lax.fori_loop(0, _CPW // _NB, _quad, 0)
    _half(_CPW - 1, (_CPW - 1) % _NB)
    _out(_CPW - 1, (_CPW - 1) % _NB).wait()


def kernel(nfeats, efeats, edge_index, W_neigh1, b_neigh1, W_edge1, b_edge1,
           W_neigh2, b_neigh2, W_edge2, b_edge2):
    ei = edge_index.astype(jnp.int32)
    dst3 = ei[1].reshape(_NW, _CPW, _C)
    psum, pdeg = _sc_scatter(efeats, dst3)
    a_tab, b_tab = _tc_linear(
        psum, pdeg,
        W_neigh2.T, b_neigh2.reshape(1, _D),
        W_edge2[:, :_D].T, W_edge2[:, _D:].T, b_edge2.reshape(1, _D))
    out = _sc_gather(a_tab, b_tab, ei)
    return out.reshape(_E, _D)
